# baseline (speedup 1.0000x reference)
import functools

import jax
import jax.numpy as jnp
from jax import lax
from jax.experimental import pallas as pl
from jax.experimental.pallas import tpu as pltpu
from jax.experimental.pallas import tpu_sc as plsc

NUM_TOKENS = 32768
HIDDEN_DIM = 768
NUM_EXPERTS = 8
TOP_K = 2

_NC = 2
_NS = 16
_L = 16
_NW = _NC * _NS
_TPW = NUM_TOKENS // _NW

_MM_BLK = 8192


def _matmul_body(hs_ref, w_ref, out_ref):
    out_ref[...] = lax.dot_general(
        w_ref[...], hs_ref[...],
        dimension_numbers=(((1,), (1,)), ((), ())),
        preferred_element_type=jnp.float32)


def _router_logits_t(hidden_states, w):
    return pl.pallas_call(
        _matmul_body,
        grid=(NUM_TOKENS // _MM_BLK,),
        in_specs=[
            pl.BlockSpec((_MM_BLK, HIDDEN_DIM), lambda i: (i, 0)),
            pl.BlockSpec((NUM_EXPERTS, HIDDEN_DIM), lambda i: (0, 0)),
        ],
        out_specs=pl.BlockSpec((NUM_EXPERTS, _MM_BLK), lambda i: (0, i)),
        out_shape=jax.ShapeDtypeStruct((NUM_EXPERTS, NUM_TOKENS), jnp.float32),
    )(hidden_states, w)


def _sc_softmax_top2_body(logits_hbm, probs_hbm, idx_hbm, lg_v, pr_v, ix_v):
    wid = lax.axis_index("s") * _NC + lax.axis_index("c")
    base = wid * _TPW
    pltpu.sync_copy(logits_hbm.at[:, pl.ds(base, _TPW)], lg_v)

    def step(g, carry):
        sl = pl.ds(g * _L, _L)
        l = [lg_v[e, sl] for e in range(NUM_EXPERTS)]
        cols = [jnp.full((_L,), e, jnp.int32) for e in range(NUM_EXPERTS)]

        m = l[0]
        for e in range(1, NUM_EXPERTS):
            m = jnp.maximum(m, l[e])
        i1 = jnp.zeros((_L,), jnp.int32)
        for e in range(NUM_EXPERTS - 1, -1, -1):
            i1 = jnp.where(l[e] == m, cols[e], i1)

        neg = jnp.full((_L,), -jnp.inf, jnp.float32)
        l2 = [jnp.where(i1 == cols[e], neg, l[e]) for e in range(NUM_EXPERTS)]
        m2 = l2[0]
        for e in range(1, NUM_EXPERTS):
            m2 = jnp.maximum(m2, l2[e])
        i2 = jnp.zeros((_L,), jnp.int32)
        for e in range(NUM_EXPERTS - 1, -1, -1):
            i2 = jnp.where(l2[e] == m2, cols[e], i2)

        z = jnp.exp(l[0] - m)
        for e in range(1, NUM_EXPERTS):
            z = z + jnp.exp(l[e] - m)
        p1 = 1.0 / z
        p2 = jnp.exp(m2 - m) * p1

        pr_v[0, sl] = p1
        pr_v[1, sl] = p2
        ix_v[0, sl] = i1
        ix_v[1, sl] = i2
        return carry

    lax.fori_loop(0, _TPW // _L, step, 0)

    pltpu.sync_copy(pr_v, probs_hbm.at[:, pl.ds(base, _TPW)])
    pltpu.sync_copy(ix_v, idx_hbm.at[:, pl.ds(base, _TPW)])


@functools.lru_cache(maxsize=1)
def _sc_softmax_top2():
    return pl.kernel(
        _sc_softmax_top2_body,
        out_type=(
            jax.ShapeDtypeStruct((TOP_K, NUM_TOKENS), jnp.float32),
            jax.ShapeDtypeStruct((TOP_K, NUM_TOKENS), jnp.int32),
        ),
        mesh=plsc.VectorSubcoreMesh(core_axis_name="c", subcore_axis_name="s",
                                    num_cores=_NC, num_subcores=_NS),
        scratch_types=[
            pltpu.VMEM((NUM_EXPERTS, _TPW), jnp.float32),
            pltpu.VMEM((TOP_K, _TPW), jnp.float32),
            pltpu.VMEM((TOP_K, _TPW), jnp.int32),
        ],
        compiler_params=pltpu.CompilerParams(needs_layout_passes=False,
                                             skip_device_barrier=True),
    )


def kernel(hidden_states, W):
    logits_t = _router_logits_t(hidden_states, W)
    probs_t, idx_t = _sc_softmax_top2()(logits_t)
    return (probs_t.T, idx_t.T)


# speedup vs baseline: 1.0474x; 1.0474x over previous
import functools

import jax
import jax.numpy as jnp
from jax import lax
from jax.experimental import pallas as pl
from jax.experimental.pallas import tpu as pltpu
from jax.experimental.pallas import tpu_sc as plsc

NUM_TOKENS = 32768
HIDDEN_DIM = 768
NUM_EXPERTS = 8
TOP_K = 2

_NC = 2
_NS = 16
_L = 16
_NW = _NC * _NS
_TPW = NUM_TOKENS // _NW

_MM_BLK = 2048


def _matmul_body(hs_ref, w_ref, out_ref):
    out_ref[...] = lax.dot_general(
        w_ref[...], hs_ref[...],
        dimension_numbers=(((1,), (1,)), ((), ())),
        preferred_element_type=jnp.float32)


def _router_logits_t(hidden_states, w):
    return pl.pallas_call(
        _matmul_body,
        grid=(NUM_TOKENS // _MM_BLK,),
        in_specs=[
            pl.BlockSpec((_MM_BLK, HIDDEN_DIM), lambda i: (i, 0)),
            pl.BlockSpec((NUM_EXPERTS, HIDDEN_DIM), lambda i: (0, 0)),
        ],
        out_specs=pl.BlockSpec((NUM_EXPERTS, _MM_BLK), lambda i: (0, i)),
        out_shape=jax.ShapeDtypeStruct((NUM_EXPERTS, NUM_TOKENS), jnp.float32),
    )(hidden_states, w)


def _sc_softmax_top2_body(logits_hbm, probs_hbm, idx_hbm, lg_v, pr_v, ix_v):
    wid = lax.axis_index("s") * _NC + lax.axis_index("c")
    base = wid * _TPW
    pltpu.sync_copy(logits_hbm.at[:, pl.ds(base, _TPW)], lg_v)

    cols = [jnp.full((_L,), e, jnp.int32) for e in range(NUM_EXPERTS)]
    neg = jnp.full((_L,), -jnp.inf, jnp.float32)

    def argmax_tree(vals):
        pairs = [(vals[e], cols[e]) for e in range(NUM_EXPERTS)]
        while len(pairs) > 1:
            nxt = []
            for (va, ia), (vb, ib) in zip(pairs[0::2], pairs[1::2]):
                keep = va >= vb
                nxt.append((jnp.where(keep, va, vb), jnp.where(keep, ia, ib)))
            pairs = nxt
        return pairs[0]

    @plsc.parallel_loop(0, _TPW // _L, unroll=2)
    def step(g):
        sl = pl.ds(g * _L, _L)
        l = [lg_v[e, sl] for e in range(NUM_EXPERTS)]

        m, i1 = argmax_tree(l)
        l2 = [jnp.where(i1 == cols[e], neg, l[e]) for e in range(NUM_EXPERTS)]
        m2, i2 = argmax_tree(l2)

        x = [jnp.exp(l[e] - m) for e in range(NUM_EXPERTS)]
        z = ((x[0] + x[1]) + (x[2] + x[3])) + ((x[4] + x[5]) + (x[6] + x[7]))
        p1 = 1.0 / z
        p2 = jnp.exp(m2 - m) * p1

        pr_v[0, sl] = p1
        pr_v[1, sl] = p2
        ix_v[0, sl] = i1
        ix_v[1, sl] = i2

    pltpu.sync_copy(pr_v, probs_hbm.at[:, pl.ds(base, _TPW)])
    pltpu.sync_copy(ix_v, idx_hbm.at[:, pl.ds(base, _TPW)])


@functools.lru_cache(maxsize=1)
def _sc_softmax_top2():
    return pl.kernel(
        _sc_softmax_top2_body,
        out_type=(
            jax.ShapeDtypeStruct((TOP_K, NUM_TOKENS), jnp.float32),
            jax.ShapeDtypeStruct((TOP_K, NUM_TOKENS), jnp.int32),
        ),
        mesh=plsc.VectorSubcoreMesh(core_axis_name="c", subcore_axis_name="s",
                                    num_cores=_NC, num_subcores=_NS),
        scratch_types=[
            pltpu.VMEM((NUM_EXPERTS, _TPW), jnp.float32),
            pltpu.VMEM((TOP_K, _TPW), jnp.float32),
            pltpu.VMEM((TOP_K, _TPW), jnp.int32),
        ],
        compiler_params=pltpu.CompilerParams(needs_layout_passes=False),
    )


def kernel(hidden_states, W):
    logits_t = _router_logits_t(hidden_states, W)
    probs_t, idx_t = _sc_softmax_top2()(logits_t)
    return (probs_t.T, idx_t.T)


# speedup vs baseline: 1.0544x; 1.0067x over previous
import functools

import jax
import jax.numpy as jnp
from jax import lax
from jax.experimental import pallas as pl
from jax.experimental.pallas import tpu as pltpu
from jax.experimental.pallas import tpu_sc as plsc

NUM_TOKENS = 32768
HIDDEN_DIM = 768
NUM_EXPERTS = 8
TOP_K = 2

_NC = 1
_NS = 16
_L = 16
_NW = _NC * _NS
_TPW = NUM_TOKENS // _NW

_MM_BLK = 4096


def _matmul_body(hs_ref, w_ref, out_ref):
    out_ref[...] = lax.dot_general(
        w_ref[...], hs_ref[...],
        dimension_numbers=(((1,), (1,)), ((), ())),
        preferred_element_type=jnp.float32)


def _router_logits_t(hidden_states, w):
    return pl.pallas_call(
        _matmul_body,
        grid=(NUM_TOKENS // _MM_BLK,),
        in_specs=[
            pl.BlockSpec((_MM_BLK, HIDDEN_DIM), lambda i: (i, 0)),
            pl.BlockSpec((NUM_EXPERTS, HIDDEN_DIM), lambda i: (0, 0)),
        ],
        out_specs=pl.BlockSpec((NUM_EXPERTS, _MM_BLK), lambda i: (0, i)),
        out_shape=jax.ShapeDtypeStruct((NUM_EXPERTS, NUM_TOKENS), jnp.float32),
    )(hidden_states, w)


def _sc_softmax_top2_body(logits_hbm, probs_hbm, idx_hbm, lg_v, pr_v, ix_v):
    wid = lax.axis_index("s") * _NC + lax.axis_index("c")
    base = wid * _TPW
    pltpu.sync_copy(logits_hbm.at[:, pl.ds(base, _TPW)], lg_v)

    cols = [jnp.full((_L,), e, jnp.int32) for e in range(NUM_EXPERTS)]
    neg = jnp.full((_L,), -jnp.inf, jnp.float32)

    def argmax_tree(vals):
        pairs = [(vals[e], cols[e]) for e in range(NUM_EXPERTS)]
        while len(pairs) > 1:
            nxt = []
            for (va, ia), (vb, ib) in zip(pairs[0::2], pairs[1::2]):
                keep = va >= vb
                nxt.append((jnp.where(keep, va, vb), jnp.where(keep, ia, ib)))
            pairs = nxt
        return pairs[0]

    @plsc.parallel_loop(0, _TPW // _L, unroll=2)
    def step(g):
        sl = pl.ds(g * _L, _L)
        l = [lg_v[e, sl] for e in range(NUM_EXPERTS)]

        m, i1 = argmax_tree(l)
        l2 = [jnp.where(i1 == cols[e], neg, l[e]) for e in range(NUM_EXPERTS)]
        m2, i2 = argmax_tree(l2)

        x = [jnp.exp(l[e] - m) for e in range(NUM_EXPERTS)]
        z = ((x[0] + x[1]) + (x[2] + x[3])) + ((x[4] + x[5]) + (x[6] + x[7]))
        p1 = 1.0 / z
        p2 = jnp.exp(m2 - m) * p1

        pr_v[0, sl] = p1
        pr_v[1, sl] = p2
        ix_v[0, sl] = i1
        ix_v[1, sl] = i2

    pltpu.sync_copy(pr_v, probs_hbm.at[:, pl.ds(base, _TPW)])
    pltpu.sync_copy(ix_v, idx_hbm.at[:, pl.ds(base, _TPW)])


@functools.lru_cache(maxsize=1)
def _sc_softmax_top2():
    return pl.kernel(
        _sc_softmax_top2_body,
        out_type=(
            jax.ShapeDtypeStruct((TOP_K, NUM_TOKENS), jnp.float32),
            jax.ShapeDtypeStruct((TOP_K, NUM_TOKENS), jnp.int32),
        ),
        mesh=plsc.VectorSubcoreMesh(core_axis_name="c", subcore_axis_name="s",
                                    num_cores=_NC, num_subcores=_NS),
        scratch_types=[
            pltpu.VMEM((NUM_EXPERTS, _TPW), jnp.float32),
            pltpu.VMEM((TOP_K, _TPW), jnp.float32),
            pltpu.VMEM((TOP_K, _TPW), jnp.int32),
        ],
        compiler_params=pltpu.CompilerParams(needs_layout_passes=False),
    )


def kernel(hidden_states, W):
    logits_t = _router_logits_t(hidden_states, W)
    probs_t, idx_t = _sc_softmax_top2()(logits_t)
    return (probs_t.T, idx_t.T)
